# baseline (speedup 1.0000x reference)
import jax
import jax.numpy as jnp
from jax import lax
from jax.experimental import pallas as pl
from jax.experimental.pallas import tpu as pltpu
from jax.experimental.pallas import tpu_sc as plsc

N = 8192
M = 8192
L = 16
NC = 2
NS = 16
W = NC * NS
QPW = N // W
U = 8
NJ = M // L

_INF = float("inf")
_IBIG = 2 ** 30

_SC_MESH = plsc.VectorSubcoreMesh(
    core_axis_name="c", subcore_axis_name="s",
    num_cores=NC, num_subcores=NS,
)


def _bf16_round(v):
    u = plsc.bitcast(v, jnp.int32)
    odd = lax.shift_right_logical(u, 16) & 1
    r = (u + 32767 + odd) & jnp.int32(-65536)
    return plsc.bitcast(r, jnp.float32)


def _search_body(xb_hbm, nx_hbm, yb_hbm, ny_hbm,
                 rcrit_hbm, ridx_hbm, ccrit_hbm, cidx_hbm,
                 xbv, nxv, ybv, nyv, ccv, civ, rcv, riv):
    wid = lax.axis_index("s") * NC + lax.axis_index("c")
    pltpu.sync_copy(xb_hbm.at[wid], xbv)
    pltpu.sync_copy(nx_hbm.at[wid], nxv)
    pltpu.sync_copy(yb_hbm, ybv)
    pltpu.sync_copy(ny_hbm, nyv)

    def round_x(k, carry):
        o = k * L
        xbv[pl.ds(o, L)] = _bf16_round(xbv[pl.ds(o, L)])
        return carry

    lax.fori_loop(0, QPW * 3, round_x, 0)

    def round_y(k, carry):
        o = k * L
        ybv[pl.ds(o, L)] = _bf16_round(ybv[pl.ds(o, L)])
        return carry

    lax.fori_loop(0, 3 * M // L, round_y, 0)

    def init_j(g, carry):
        ccv[pl.ds(g * L, L)] = jnp.full((L,), _INF, jnp.float32)
        civ[pl.ds(g * L, L)] = jnp.zeros((L,), jnp.int32)
        return carry

    lax.fori_loop(0, NJ, init_j, 0)

    lane = lax.iota(jnp.int32, L)
    i_base = wid * QPW

    def iblock(ib, carry):
        q0 = ib * U
        xs = [tuple(xbv[pl.ds(((q0 + u) * 3 + c) * L, L)] for c in range(3))
              for u in range(U)]
        nxs = [nxv[pl.ds((q0 + u) * L, L)] for u in range(U)]
        ivs = [jnp.full((L,), i_base + q0 + u, jnp.int32) for u in range(U)]

        def jgroup(g, rc_ri):
            o = g * L
            yb0 = ybv[pl.ds(o, L)]
            yb1 = ybv[pl.ds(M + o, L)]
            yb2 = ybv[pl.ds(2 * M + o, L)]
            ny = nyv[pl.ds(o, L)]
            cc = ccv[pl.ds(o, L)]
            ci = civ[pl.ds(o, L)]
            jv = o + lane
            new = []
            for u in range(U):
                dot = xs[u][0] * yb0 + xs[u][1] * yb1 + xs[u][2] * yb2
                t2 = dot + dot
                r = ny - t2
                rm = r < rc_ri[u][0]
                brc = jnp.where(rm, r, rc_ri[u][0])
                bri = jnp.where(rm, jv, rc_ri[u][1])
                new.append((brc, bri))
                c = nxs[u] - t2
                cm = c < cc
                cc = jnp.where(cm, c, cc)
                ci = jnp.where(cm, ivs[u], ci)
            ccv[pl.ds(o, L)] = cc
            civ[pl.ds(o, L)] = ci
            return tuple(new)

        init = tuple((jnp.full((L,), _INF, jnp.float32),
                      jnp.zeros((L,), jnp.int32)) for _ in range(U))
        rc_ri = lax.fori_loop(0, NJ, jgroup, init)
        for u in range(U):
            rcv[pl.ds((q0 + u) * L, L)] = rc_ri[u][0]
            riv[pl.ds((q0 + u) * L, L)] = rc_ri[u][1]
        return carry

    lax.fori_loop(0, QPW // U, iblock, 0)

    pltpu.sync_copy(rcv, rcrit_hbm.at[wid])
    pltpu.sync_copy(riv, ridx_hbm.at[wid])
    pltpu.sync_copy(ccv, ccrit_hbm.at[wid])
    pltpu.sync_copy(civ, cidx_hbm.at[wid])


_sc_search = pl.kernel(
    _search_body,
    out_type=(
        jax.ShapeDtypeStruct((W, QPW * L), jnp.float32),
        jax.ShapeDtypeStruct((W, QPW * L), jnp.int32),
        jax.ShapeDtypeStruct((W, M), jnp.float32),
        jax.ShapeDtypeStruct((W, M), jnp.int32),
    ),
    mesh=_SC_MESH,
    scratch_types=[
        pltpu.VMEM((QPW * 3 * L,), jnp.float32),
        pltpu.VMEM((QPW * L,), jnp.float32),
        pltpu.VMEM((3 * M,), jnp.float32),
        pltpu.VMEM((M,), jnp.float32),
        pltpu.VMEM((M,), jnp.float32),
        pltpu.VMEM((M,), jnp.int32),
        pltpu.VMEM((QPW * L,), jnp.float32),
        pltpu.VMEM((QPW * L,), jnp.int32),
    ],
    compiler_params=pltpu.CompilerParams(needs_layout_passes=False),
)


def _argmerge_body(rc_ref, ri_ref, cc_ref, ci_ref, xnn_ref, ynn_ref):
    rc = rc_ref[...]
    rmin = jnp.min(rc, axis=1, keepdims=True)
    xnn = jnp.min(jnp.where(rc == rmin, ri_ref[...], _IBIG), axis=1)
    xnn_ref[...] = xnn.reshape(1, N)
    cc = cc_ref[...]
    cmin = jnp.min(cc, axis=0, keepdims=True)
    ynn = jnp.min(jnp.where(cc == cmin, ci_ref[...], _IBIG), axis=0)
    ynn_ref[...] = ynn.reshape(1, M)


_tc_argmerge = pl.pallas_call(
    _argmerge_body,
    out_shape=(
        jax.ShapeDtypeStruct((1, N), jnp.int32),
        jax.ShapeDtypeStruct((1, M), jnp.int32),
    ),
)


def _gather_body(xt_hbm, yt_hbm, xnn_hbm, ynn_hbm, rsum_hbm, csum_hbm,
                 xtv, ytv, ixv, iyv, rsv, csv):
    wid = lax.axis_index("s") * NC + lax.axis_index("c")
    base = wid * QPW
    pltpu.sync_copy(xt_hbm, xtv)
    pltpu.sync_copy(yt_hbm, ytv)
    pltpu.sync_copy(xnn_hbm.at[pl.ds(base, QPW)], ixv)
    pltpu.sync_copy(ynn_hbm.at[pl.ds(base, QPW)], iyv)

    def group(k, acc):
        accr, accc = acc
        o = k * L
        jn = ixv[pl.ds(o, L)]
        g0 = plsc.load_gather(ytv, [jn])
        g1 = plsc.load_gather(ytv, [jn + M])
        g2 = plsc.load_gather(ytv, [jn + 2 * M])
        x0 = xtv[pl.ds(base + o, L)]
        x1 = xtv[pl.ds(N + base + o, L)]
        x2 = xtv[pl.ds(2 * N + base + o, L)]
        d0 = x0 - g0
        d1 = x1 - g1
        d2 = x2 - g2
        accr = accr + (d0 * d0 + d1 * d1 + d2 * d2)
        im = iyv[pl.ds(o, L)]
        h0 = plsc.load_gather(xtv, [im])
        h1 = plsc.load_gather(xtv, [im + N])
        h2 = plsc.load_gather(xtv, [im + 2 * N])
        y0 = ytv[pl.ds(base + o, L)]
        y1 = ytv[pl.ds(M + base + o, L)]
        y2 = ytv[pl.ds(2 * M + base + o, L)]
        e0 = y0 - h0
        e1 = y1 - h1
        e2 = y2 - h2
        accc = accc + (e0 * e0 + e1 * e1 + e2 * e2)
        return accr, accc

    zero = jnp.zeros((L,), jnp.float32)
    accr, accc = lax.fori_loop(0, QPW // L, group, (zero, zero))
    rsv[...] = accr
    csv[...] = accc
    pltpu.sync_copy(rsv, rsum_hbm.at[wid])
    pltpu.sync_copy(csv, csum_hbm.at[wid])


_sc_gather = pl.kernel(
    _gather_body,
    out_type=(
        jax.ShapeDtypeStruct((W, L), jnp.float32),
        jax.ShapeDtypeStruct((W, L), jnp.float32),
    ),
    mesh=_SC_MESH,
    scratch_types=[
        pltpu.VMEM((3 * N,), jnp.float32),
        pltpu.VMEM((3 * M,), jnp.float32),
        pltpu.VMEM((QPW,), jnp.int32),
        pltpu.VMEM((QPW,), jnp.int32),
        pltpu.VMEM((L,), jnp.float32),
        pltpu.VMEM((L,), jnp.float32),
    ],
    compiler_params=pltpu.CompilerParams(needs_layout_passes=False),
)


def _finish_body(rs_ref, cs_ref, out_ref):
    out_ref[0, 0] = 0.5 * (jnp.sum(rs_ref[...]) / N + jnp.sum(cs_ref[...]) / M)


_tc_finish = pl.pallas_call(
    _finish_body,
    out_shape=jax.ShapeDtypeStruct((1, 1), jnp.float32),
    out_specs=pl.BlockSpec(memory_space=pltpu.MemorySpace.SMEM),
)


@jax.jit
def kernel(x, y):
    nx = jnp.sum(x * x, axis=1)
    ny = jnp.sum(y * y, axis=1)
    xbr = jnp.broadcast_to(x.reshape(W, QPW, 3, 1), (W, QPW, 3, L))
    xbr = xbr.reshape(W, QPW * 3 * L)
    nxr = jnp.broadcast_to(nx.reshape(W, QPW, 1), (W, QPW, L)).reshape(W, QPW * L)
    ybt = y.T.reshape(3 * M)

    rcrit, ridx, ccrit, cidx = _sc_search(xbr, nxr, ybt, ny)
    xnn, ynn = _tc_argmerge(rcrit.reshape(N, L), ridx.reshape(N, L),
                            ccrit, cidx)
    rsum, csum = _sc_gather(x.T.reshape(3 * N), y.T.reshape(3 * M),
                            xnn.reshape(N), ynn.reshape(M))
    out = _tc_finish(rsum, csum)
    return out[0, 0]


# speedup vs baseline: 1.2867x; 1.2867x over previous
import jax
import jax.numpy as jnp
from jax import lax
from jax.experimental import pallas as pl
from jax.experimental.pallas import tpu as pltpu
from jax.experimental.pallas import tpu_sc as plsc

N = 8192
M = 8192
L = 16
NC = 2
NS = 16
W = NC * NS
QPW = N // W

TM = 256
TN = 512
NIB = N // TM
NJB = M // TN

_INF = float("inf")


def _search_body(x8_ref, y8t_ref, nyh_ref, nxh_ref, xnn_ref, ynn_ref,
                 rmin_s, ridx_s, cmin_s, cidx_s):
    i = pl.program_id(0)
    j = pl.program_id(1)

    @pl.when(jnp.logical_and(i == 0, j == 0))
    def _():
        cmin_s[...] = jnp.full((M,), _INF, jnp.float32)
        cidx_s[...] = jnp.zeros((M,), jnp.int32)

    @pl.when(j == 0)
    def _():
        rmin_s[...] = jnp.full((TM,), _INF, jnp.float32)
        ridx_s[...] = jnp.zeros((TM,), jnp.int32)

    xb = x8_ref[...].astype(jnp.bfloat16)
    yb = y8t_ref[...].astype(jnp.bfloat16)
    g = jnp.dot(xb, yb, preferred_element_type=jnp.float32)

    crit_r = nyh_ref[...] - g
    tmin = jnp.min(crit_r, axis=1)
    targ = jnp.argmin(crit_r, axis=1).astype(jnp.int32) + j * TN
    better = tmin < rmin_s[...]
    rmin_s[...] = jnp.where(better, tmin, rmin_s[...])
    ridx_s[...] = jnp.where(better, targ, ridx_s[...])

    crit_c = nxh_ref[...] - g
    cmin_t = jnp.min(crit_c, axis=0)
    carg_t = jnp.argmin(crit_c, axis=0).astype(jnp.int32) + i * TM
    oldc = cmin_s[pl.ds(j * TN, TN)]
    oldi = cidx_s[pl.ds(j * TN, TN)]
    cbetter = cmin_t < oldc
    cmin_s[pl.ds(j * TN, TN)] = jnp.where(cbetter, cmin_t, oldc)
    cidx_s[pl.ds(j * TN, TN)] = jnp.where(cbetter, carg_t, oldi)

    @pl.when(j == NJB - 1)
    def _():
        xnn_ref[...] = ridx_s[...].reshape(1, TM)

    @pl.when(jnp.logical_and(i == NIB - 1, j == NJB - 1))
    def _():
        ynn_ref[...] = cidx_s[...].reshape(1, M)


_tc_search = pl.pallas_call(
    _search_body,
    grid=(NIB, NJB),
    in_specs=[
        pl.BlockSpec((TM, 8), lambda i, j: (i, 0)),
        pl.BlockSpec((8, TN), lambda i, j: (0, j)),
        pl.BlockSpec((1, TN), lambda i, j: (0, j)),
        pl.BlockSpec((TM, 1), lambda i, j: (i, 0)),
    ],
    out_specs=(
        pl.BlockSpec((1, TM), lambda i, j: (0, i)),
        pl.BlockSpec((1, M), lambda i, j: (0, 0)),
    ),
    out_shape=(
        jax.ShapeDtypeStruct((1, N), jnp.int32),
        jax.ShapeDtypeStruct((1, M), jnp.int32),
    ),
    scratch_shapes=[
        pltpu.VMEM((TM,), jnp.float32),
        pltpu.VMEM((TM,), jnp.int32),
        pltpu.VMEM((M,), jnp.float32),
        pltpu.VMEM((M,), jnp.int32),
    ],
)


_SC_MESH = plsc.VectorSubcoreMesh(
    core_axis_name="c", subcore_axis_name="s",
    num_cores=NC, num_subcores=NS,
)


def _gather_body(xt_hbm, yt_hbm, xnn_hbm, ynn_hbm, rsum_hbm, csum_hbm,
                 xtv, ytv, ixv, iyv, rsv, csv):
    wid = lax.axis_index("s") * NC + lax.axis_index("c")
    base = wid * QPW
    pltpu.sync_copy(xt_hbm, xtv)
    pltpu.sync_copy(yt_hbm, ytv)
    pltpu.sync_copy(xnn_hbm.at[pl.ds(base, QPW)], ixv)
    pltpu.sync_copy(ynn_hbm.at[pl.ds(base, QPW)], iyv)

    def group(k, acc):
        accr, accc = acc
        o = k * L
        jn = ixv[pl.ds(o, L)]
        g0 = plsc.load_gather(ytv, [jn])
        g1 = plsc.load_gather(ytv, [jn + M])
        g2 = plsc.load_gather(ytv, [jn + 2 * M])
        x0 = xtv[pl.ds(base + o, L)]
        x1 = xtv[pl.ds(N + base + o, L)]
        x2 = xtv[pl.ds(2 * N + base + o, L)]
        d0 = x0 - g0
        d1 = x1 - g1
        d2 = x2 - g2
        accr = accr + (d0 * d0 + d1 * d1 + d2 * d2)
        im = iyv[pl.ds(o, L)]
        h0 = plsc.load_gather(xtv, [im])
        h1 = plsc.load_gather(xtv, [im + N])
        h2 = plsc.load_gather(xtv, [im + 2 * N])
        y0 = ytv[pl.ds(base + o, L)]
        y1 = ytv[pl.ds(M + base + o, L)]
        y2 = ytv[pl.ds(2 * M + base + o, L)]
        e0 = y0 - h0
        e1 = y1 - h1
        e2 = y2 - h2
        accc = accc + (e0 * e0 + e1 * e1 + e2 * e2)
        return accr, accc

    zero = jnp.zeros((L,), jnp.float32)
    accr, accc = lax.fori_loop(0, QPW // L, group, (zero, zero))
    rsv[...] = accr
    csv[...] = accc
    pltpu.sync_copy(rsv, rsum_hbm.at[wid])
    pltpu.sync_copy(csv, csum_hbm.at[wid])


_sc_gather = pl.kernel(
    _gather_body,
    out_type=(
        jax.ShapeDtypeStruct((W, L), jnp.float32),
        jax.ShapeDtypeStruct((W, L), jnp.float32),
    ),
    mesh=_SC_MESH,
    scratch_types=[
        pltpu.VMEM((3 * N,), jnp.float32),
        pltpu.VMEM((3 * M,), jnp.float32),
        pltpu.VMEM((QPW,), jnp.int32),
        pltpu.VMEM((QPW,), jnp.int32),
        pltpu.VMEM((L,), jnp.float32),
        pltpu.VMEM((L,), jnp.float32),
    ],
    compiler_params=pltpu.CompilerParams(needs_layout_passes=False),
)


def _finish_body(rs_ref, cs_ref, out_ref):
    out_ref[0, 0] = 0.5 * (jnp.sum(rs_ref[...]) / N + jnp.sum(cs_ref[...]) / M)


_tc_finish = pl.pallas_call(
    _finish_body,
    out_shape=jax.ShapeDtypeStruct((1, 1), jnp.float32),
    out_specs=pl.BlockSpec(memory_space=pltpu.MemorySpace.SMEM),
)


@jax.jit
def kernel(x, y):
    nx = jnp.sum(x * x, axis=1)
    ny = jnp.sum(y * y, axis=1)
    x8 = jnp.concatenate([x, jnp.zeros((N, 5), jnp.float32)], axis=1)
    y8t = jnp.concatenate([y, jnp.zeros((M, 5), jnp.float32)], axis=1).T
    xnn, ynn = _tc_search(x8, y8t, (0.5 * ny).reshape(1, M),
                          (0.5 * nx).reshape(N, 1))
    rsum, csum = _sc_gather(x.T.reshape(3 * N), y.T.reshape(3 * M),
                            xnn.reshape(N), ynn.reshape(M))
    out = _tc_finish(rsum, csum)
    return out[0, 0]


# speedup vs baseline: 1.6703x; 1.2981x over previous
import jax
import jax.numpy as jnp
from jax import lax
from jax.experimental import pallas as pl
from jax.experimental.pallas import tpu as pltpu
from jax.experimental.pallas import tpu_sc as plsc

N = 8192
M = 8192
L = 16
NC = 2
NS = 16
W = NC * NS
QPW = N // W

TM = 256
TN = 512
NIB = N // TM
NJB = M // TN

_INF = float("inf")


NCH = TN // 128
NRS = TM // 8
_IBIG = 1 << 30


def _lexmin(a, b):
    av, ai = a
    bv, bi = b
    t = bv < av
    return jnp.where(t, bv, av), jnp.where(t, bi, ai)


def _tree_fold(items):
    while len(items) > 1:
        nxt = [_lexmin(items[k], items[k + 1])
               for k in range(0, len(items) - 1, 2)]
        if len(items) % 2:
            nxt.append(items[-1])
        items = nxt
    return items[0]


def _search_body(x8_ref, y8t_ref, nyh_ref, nxh_ref, xnn_ref, ynn_ref,
                 rmin_s, ridx_s, cmin_s, cgid_s):
    i = pl.program_id(0)
    j = pl.program_id(1)

    xb = x8_ref[...].astype(jnp.bfloat16)
    yb = y8t_ref[...].astype(jnp.bfloat16)
    g = jnp.dot(xb, yb, preferred_element_type=jnp.float32)
    nyh = nyh_ref[...]
    nxh = nxh_ref[...]

    ritems = [(nyh[:, c * 128:(c + 1) * 128] - g[:, c * 128:(c + 1) * 128],
               jnp.full((TM, 128), j * NCH + c, jnp.int32))
              for c in range(NCH)]
    rv, ri = _tree_fold(ritems)

    @pl.when(j == 0)
    def _():
        rmin_s[...] = rv
        ridx_s[...] = ri

    @pl.when(j > 0)
    def _():
        mv, mi = _lexmin((rmin_s[...], ridx_s[...]), (rv, ri))
        rmin_s[...] = mv
        ridx_s[...] = mi

    citems = [(nxh[t * 8:(t + 1) * 8, :] - g[t * 8:(t + 1) * 8, :],
               jnp.full((8, TN), i * NRS + t, jnp.int32))
              for t in range(NRS)]
    cv, ci = _tree_fold(citems)

    @pl.when(i == 0)
    def _():
        cmin_s[:, pl.ds(j * TN, TN)] = cv
        cgid_s[:, pl.ds(j * TN, TN)] = ci

    @pl.when(i > 0)
    def _():
        mv, mi = _lexmin((cmin_s[:, pl.ds(j * TN, TN)],
                          cgid_s[:, pl.ds(j * TN, TN)]), (cv, ci))
        cmin_s[:, pl.ds(j * TN, TN)] = mv
        cgid_s[:, pl.ds(j * TN, TN)] = mi

    @pl.when(j == NJB - 1)
    def _():
        vals = rmin_s[...]
        fidx = ridx_s[...] * 128 + lax.broadcasted_iota(jnp.int32, (TM, 128), 1)
        best = jnp.min(vals, axis=1, keepdims=True)
        cand = jnp.where(vals == best, fidx, _IBIG)
        xnn_ref[...] = jnp.min(cand, axis=1).reshape(1, TM)

    @pl.when(i == NIB - 1)
    def _():
        vals = cmin_s[:, pl.ds(j * TN, TN)]
        frow = (cgid_s[:, pl.ds(j * TN, TN)] * 8
                + lax.broadcasted_iota(jnp.int32, (8, TN), 0))
        best = jnp.min(vals, axis=0, keepdims=True)
        cand = jnp.where(vals == best, frow, _IBIG)
        ynn_ref[...] = jnp.min(cand, axis=0).reshape(1, TN)


_tc_search = pl.pallas_call(
    _search_body,
    grid=(NIB, NJB),
    in_specs=[
        pl.BlockSpec((TM, 8), lambda i, j: (i, 0)),
        pl.BlockSpec((8, TN), lambda i, j: (0, j)),
        pl.BlockSpec((1, TN), lambda i, j: (0, j)),
        pl.BlockSpec((TM, 1), lambda i, j: (i, 0)),
    ],
    out_specs=(
        pl.BlockSpec((1, TM), lambda i, j: (0, i)),
        pl.BlockSpec((1, TN), lambda i, j: (0, j)),
    ),
    out_shape=(
        jax.ShapeDtypeStruct((1, N), jnp.int32),
        jax.ShapeDtypeStruct((1, M), jnp.int32),
    ),
    scratch_shapes=[
        pltpu.VMEM((TM, 128), jnp.float32),
        pltpu.VMEM((TM, 128), jnp.int32),
        pltpu.VMEM((8, M), jnp.float32),
        pltpu.VMEM((8, M), jnp.int32),
    ],
)


_SC_MESH = plsc.VectorSubcoreMesh(
    core_axis_name="c", subcore_axis_name="s",
    num_cores=NC, num_subcores=NS,
)


def _gather_body(xt_hbm, yt_hbm, xnn_hbm, ynn_hbm, rsum_hbm, csum_hbm,
                 xtv, ytv, ixv, iyv, rsv, csv):
    wid = lax.axis_index("s") * NC + lax.axis_index("c")
    base = wid * QPW
    pltpu.sync_copy(xt_hbm, xtv)
    pltpu.sync_copy(yt_hbm, ytv)
    pltpu.sync_copy(xnn_hbm.at[pl.ds(base, QPW)], ixv)
    pltpu.sync_copy(ynn_hbm.at[pl.ds(base, QPW)], iyv)

    def group(k, acc):
        accr, accc = acc
        o = k * L
        jn = ixv[pl.ds(o, L)]
        g0 = plsc.load_gather(ytv, [jn])
        g1 = plsc.load_gather(ytv, [jn + M])
        g2 = plsc.load_gather(ytv, [jn + 2 * M])
        x0 = xtv[pl.ds(base + o, L)]
        x1 = xtv[pl.ds(N + base + o, L)]
        x2 = xtv[pl.ds(2 * N + base + o, L)]
        d0 = x0 - g0
        d1 = x1 - g1
        d2 = x2 - g2
        accr = accr + (d0 * d0 + d1 * d1 + d2 * d2)
        im = iyv[pl.ds(o, L)]
        h0 = plsc.load_gather(xtv, [im])
        h1 = plsc.load_gather(xtv, [im + N])
        h2 = plsc.load_gather(xtv, [im + 2 * N])
        y0 = ytv[pl.ds(base + o, L)]
        y1 = ytv[pl.ds(M + base + o, L)]
        y2 = ytv[pl.ds(2 * M + base + o, L)]
        e0 = y0 - h0
        e1 = y1 - h1
        e2 = y2 - h2
        accc = accc + (e0 * e0 + e1 * e1 + e2 * e2)
        return accr, accc

    zero = jnp.zeros((L,), jnp.float32)
    accr, accc = lax.fori_loop(0, QPW // L, group, (zero, zero))
    rsv[...] = accr
    csv[...] = accc
    pltpu.sync_copy(rsv, rsum_hbm.at[wid])
    pltpu.sync_copy(csv, csum_hbm.at[wid])


_sc_gather = pl.kernel(
    _gather_body,
    out_type=(
        jax.ShapeDtypeStruct((W, L), jnp.float32),
        jax.ShapeDtypeStruct((W, L), jnp.float32),
    ),
    mesh=_SC_MESH,
    scratch_types=[
        pltpu.VMEM((3 * N,), jnp.float32),
        pltpu.VMEM((3 * M,), jnp.float32),
        pltpu.VMEM((QPW,), jnp.int32),
        pltpu.VMEM((QPW,), jnp.int32),
        pltpu.VMEM((L,), jnp.float32),
        pltpu.VMEM((L,), jnp.float32),
    ],
    compiler_params=pltpu.CompilerParams(needs_layout_passes=False),
)


def _finish_body(rs_ref, cs_ref, out_ref):
    out_ref[0, 0] = 0.5 * (jnp.sum(rs_ref[...]) / N + jnp.sum(cs_ref[...]) / M)


_tc_finish = pl.pallas_call(
    _finish_body,
    out_shape=jax.ShapeDtypeStruct((1, 1), jnp.float32),
    out_specs=pl.BlockSpec(memory_space=pltpu.MemorySpace.SMEM),
)


@jax.jit
def kernel(x, y):
    nx = jnp.sum(x * x, axis=1)
    ny = jnp.sum(y * y, axis=1)
    x8 = jnp.concatenate([x, jnp.zeros((N, 5), jnp.float32)], axis=1)
    y8t = jnp.concatenate([y, jnp.zeros((M, 5), jnp.float32)], axis=1).T
    xnn, ynn = _tc_search(x8, y8t, (0.5 * ny).reshape(1, M),
                          (0.5 * nx).reshape(N, 1))
    rsum, csum = _sc_gather(x.T.reshape(3 * N), y.T.reshape(3 * M),
                            xnn.reshape(N), ynn.reshape(M))
    out = _tc_finish(rsum, csum)
    return out[0, 0]


# speedup vs baseline: 2.4694x; 1.4784x over previous
import jax
import jax.numpy as jnp
from jax import lax
from jax.experimental import pallas as pl
from jax.experimental.pallas import tpu as pltpu
from jax.experimental.pallas import tpu_sc as plsc

N = 8192
M = 8192
L = 16
NC = 2
NS = 16
W = NC * NS
QPW = N // W

TM = 256
TN = 1024
NIB = N // TM
NJB = M // TN

_INF = float("inf")


NCH = TN // 128
NRS = TM // 8
_IBIG = 1 << 30


def _lexmin(a, b):
    av, ai = a
    bv, bi = b
    t = bv < av
    return jnp.where(t, bv, av), jnp.where(t, bi, ai)


def _tree_fold(items):
    while len(items) > 1:
        nxt = [_lexmin(items[k], items[k + 1])
               for k in range(0, len(items) - 1, 2)]
        if len(items) % 2:
            nxt.append(items[-1])
        items = nxt
    return items[0]


def _search_body(x8_ref, y8t_ref, nyh_ref, nxh_ref, rmin_o, ridx_o, ynn_ref,
                 cmin_s, cgid_s):
    i = pl.program_id(0)
    j = pl.program_id(1)

    xb = x8_ref[...].astype(jnp.bfloat16)
    yb = y8t_ref[...].astype(jnp.bfloat16)
    g = jnp.dot(xb, yb, preferred_element_type=jnp.float32)
    nyh = nyh_ref[...]
    nxh = nxh_ref[...]

    ritems = [(nyh[:, c * 128:(c + 1) * 128] - g[:, c * 128:(c + 1) * 128],
               jnp.full((TM, 128), j * NCH + c, jnp.int32))
              for c in range(NCH)]
    rv, ri = _tree_fold(ritems)

    @pl.when(j == 0)
    def _():
        rmin_o[...] = rv
        ridx_o[...] = ri

    @pl.when(j > 0)
    def _():
        mv, mi = _lexmin((rmin_o[...], ridx_o[...]), (rv, ri))
        rmin_o[...] = mv
        ridx_o[...] = mi

    citems = [(nxh[t * 8:(t + 1) * 8, :] - g[t * 8:(t + 1) * 8, :],
               jnp.full((8, TN), i * NRS + t, jnp.int32))
              for t in range(NRS)]
    cv, ci = _tree_fold(citems)

    @pl.when(i == 0)
    def _():
        cmin_s[:, pl.ds(j * TN, TN)] = cv
        cgid_s[:, pl.ds(j * TN, TN)] = ci

    @pl.when(i > 0)
    def _():
        mv, mi = _lexmin((cmin_s[:, pl.ds(j * TN, TN)],
                          cgid_s[:, pl.ds(j * TN, TN)]), (cv, ci))
        cmin_s[:, pl.ds(j * TN, TN)] = mv
        cgid_s[:, pl.ds(j * TN, TN)] = mi

    @pl.when(i == NIB - 1)
    def _():
        vals = cmin_s[:, pl.ds(j * TN, TN)]
        frow = (cgid_s[:, pl.ds(j * TN, TN)] * 8
                + lax.broadcasted_iota(jnp.int32, (8, TN), 0))
        best = jnp.min(vals, axis=0, keepdims=True)
        cand = jnp.where(vals == best, frow, _IBIG)
        ynn_ref[...] = jnp.min(cand, axis=0).reshape(1, TN)


_tc_search = pl.pallas_call(
    _search_body,
    grid=(NIB, NJB),
    in_specs=[
        pl.BlockSpec((TM, 8), lambda i, j: (i, 0)),
        pl.BlockSpec((8, TN), lambda i, j: (0, j)),
        pl.BlockSpec((1, TN), lambda i, j: (0, j)),
        pl.BlockSpec((TM, 1), lambda i, j: (i, 0)),
    ],
    out_specs=(
        pl.BlockSpec((TM, 128), lambda i, j: (i, 0)),
        pl.BlockSpec((TM, 128), lambda i, j: (i, 0)),
        pl.BlockSpec((1, TN), lambda i, j: (0, j)),
    ),
    out_shape=(
        jax.ShapeDtypeStruct((N, 128), jnp.float32),
        jax.ShapeDtypeStruct((N, 128), jnp.int32),
        jax.ShapeDtypeStruct((1, M), jnp.int32),
    ),
    scratch_shapes=[
        pltpu.VMEM((8, M), jnp.float32),
        pltpu.VMEM((8, M), jnp.int32),
    ],
)


def _rowfin_body(rmin_ref, ridx_ref, xnn_ref):
    vals = rmin_ref[...]
    fidx = ridx_ref[...] * 128 + lax.broadcasted_iota(jnp.int32, (TM, 128), 1)
    best = jnp.min(vals, axis=1, keepdims=True)
    cand = jnp.where(vals == best, fidx, _IBIG)
    xnn_ref[...] = jnp.min(cand, axis=1).reshape(1, TM)


_tc_rowfin = pl.pallas_call(
    _rowfin_body,
    grid=(NIB,),
    in_specs=[
        pl.BlockSpec((TM, 128), lambda i: (i, 0)),
        pl.BlockSpec((TM, 128), lambda i: (i, 0)),
    ],
    out_specs=pl.BlockSpec((1, TM), lambda i: (0, i)),
    out_shape=jax.ShapeDtypeStruct((1, N), jnp.int32),
)


_SC_MESH = plsc.VectorSubcoreMesh(
    core_axis_name="c", subcore_axis_name="s",
    num_cores=NC, num_subcores=NS,
)


def _gather_body(xt_hbm, yt_hbm, xnn_hbm, ynn_hbm, rsum_hbm, csum_hbm,
                 xtv, ytv, ixv, iyv, rsv, csv):
    wid = lax.axis_index("s") * NC + lax.axis_index("c")
    base = wid * QPW
    pltpu.sync_copy(xt_hbm, xtv)
    pltpu.sync_copy(yt_hbm, ytv)
    pltpu.sync_copy(xnn_hbm.at[pl.ds(base, QPW)], ixv)
    pltpu.sync_copy(ynn_hbm.at[pl.ds(base, QPW)], iyv)

    def group(k, acc):
        accr, accc = acc
        o = k * L
        jn = ixv[pl.ds(o, L)]
        g0 = plsc.load_gather(ytv, [jn])
        g1 = plsc.load_gather(ytv, [jn + M])
        g2 = plsc.load_gather(ytv, [jn + 2 * M])
        x0 = xtv[pl.ds(base + o, L)]
        x1 = xtv[pl.ds(N + base + o, L)]
        x2 = xtv[pl.ds(2 * N + base + o, L)]
        d0 = x0 - g0
        d1 = x1 - g1
        d2 = x2 - g2
        accr = accr + (d0 * d0 + d1 * d1 + d2 * d2)
        im = iyv[pl.ds(o, L)]
        h0 = plsc.load_gather(xtv, [im])
        h1 = plsc.load_gather(xtv, [im + N])
        h2 = plsc.load_gather(xtv, [im + 2 * N])
        y0 = ytv[pl.ds(base + o, L)]
        y1 = ytv[pl.ds(M + base + o, L)]
        y2 = ytv[pl.ds(2 * M + base + o, L)]
        e0 = y0 - h0
        e1 = y1 - h1
        e2 = y2 - h2
        accc = accc + (e0 * e0 + e1 * e1 + e2 * e2)
        return accr, accc

    zero = jnp.zeros((L,), jnp.float32)
    accr, accc = lax.fori_loop(0, QPW // L, group, (zero, zero))
    rsv[...] = accr
    csv[...] = accc
    pltpu.sync_copy(rsv, rsum_hbm.at[wid])
    pltpu.sync_copy(csv, csum_hbm.at[wid])


_sc_gather = pl.kernel(
    _gather_body,
    out_type=(
        jax.ShapeDtypeStruct((W, L), jnp.float32),
        jax.ShapeDtypeStruct((W, L), jnp.float32),
    ),
    mesh=_SC_MESH,
    scratch_types=[
        pltpu.VMEM((3 * N,), jnp.float32),
        pltpu.VMEM((3 * M,), jnp.float32),
        pltpu.VMEM((QPW,), jnp.int32),
        pltpu.VMEM((QPW,), jnp.int32),
        pltpu.VMEM((L,), jnp.float32),
        pltpu.VMEM((L,), jnp.float32),
    ],
    compiler_params=pltpu.CompilerParams(needs_layout_passes=False),
)


def _finish_body(rs_ref, cs_ref, out_ref):
    out_ref[0, 0] = 0.5 * (jnp.sum(rs_ref[...]) / N + jnp.sum(cs_ref[...]) / M)


_tc_finish = pl.pallas_call(
    _finish_body,
    out_shape=jax.ShapeDtypeStruct((1, 1), jnp.float32),
    out_specs=pl.BlockSpec(memory_space=pltpu.MemorySpace.SMEM),
)


@jax.jit
def kernel(x, y):
    nx = jnp.sum(x * x, axis=1)
    ny = jnp.sum(y * y, axis=1)
    x8 = jnp.concatenate([x, jnp.zeros((N, 5), jnp.float32)], axis=1)
    y8t = jnp.concatenate([y, jnp.zeros((M, 5), jnp.float32)], axis=1).T
    rmin, ridx, ynn = _tc_search(x8, y8t, (0.5 * ny).reshape(1, M),
                                 (0.5 * nx).reshape(N, 1))
    xnn = _tc_rowfin(rmin, ridx)
    rsum, csum = _sc_gather(x.T.reshape(3 * N), y.T.reshape(3 * M),
                            xnn.reshape(N), ynn.reshape(M))
    out = _tc_finish(rsum, csum)
    return out[0, 0]


# speedup vs baseline: 3.1665x; 1.2823x over previous
import jax
import jax.numpy as jnp
from jax import lax
from jax.experimental import pallas as pl
from jax.experimental.pallas import tpu as pltpu
from jax.experimental.pallas import tpu_sc as plsc

N = 8192
M = 8192
L = 16
NC = 2
NS = 16
W = NC * NS
QPW = N // W

TM = 256
TN = 2048
NIB = N // TM
NJB = M // TN

_INF = float("inf")


NCH = TN // 128
NRS = TM // 8
_IBIG = 1 << 30


def _lexmin(a, b):
    av, ai = a
    bv, bi = b
    t = bv < av
    return jnp.where(t, bv, av), jnp.where(t, bi, ai)


def _tree_fold(items):
    while len(items) > 1:
        nxt = [_lexmin(items[k], items[k + 1])
               for k in range(0, len(items) - 1, 2)]
        if len(items) % 2:
            nxt.append(items[-1])
        items = nxt
    return items[0]


def _search_body(x8_ref, y8t_ref, nyh_ref, nxh_ref, rmin_o, ridx_o, ynn_ref,
                 cmin_s, cgid_s):
    i = pl.program_id(0)
    j = pl.program_id(1)

    xb = x8_ref[...].astype(jnp.bfloat16)
    yb = y8t_ref[...].astype(jnp.bfloat16)
    g = jnp.dot(xb, yb, preferred_element_type=jnp.float32)
    nyh = nyh_ref[...]
    nxh = nxh_ref[...]

    ritems = [(nyh[:, c * 128:(c + 1) * 128] - g[:, c * 128:(c + 1) * 128],
               jnp.full((TM, 128), j * NCH + c, jnp.int32))
              for c in range(NCH)]
    rv, ri = _tree_fold(ritems)

    @pl.when(j == 0)
    def _():
        rmin_o[...] = rv
        ridx_o[...] = ri

    @pl.when(j > 0)
    def _():
        mv, mi = _lexmin((rmin_o[...], ridx_o[...]), (rv, ri))
        rmin_o[...] = mv
        ridx_o[...] = mi

    citems = [(nxh[t * 8:(t + 1) * 8, :] - g[t * 8:(t + 1) * 8, :],
               jnp.full((8, TN), i * NRS + t, jnp.int32))
              for t in range(NRS)]
    cv, ci = _tree_fold(citems)

    @pl.when(i == 0)
    def _():
        cmin_s[:, pl.ds(j * TN, TN)] = cv
        cgid_s[:, pl.ds(j * TN, TN)] = ci

    @pl.when(i > 0)
    def _():
        mv, mi = _lexmin((cmin_s[:, pl.ds(j * TN, TN)],
                          cgid_s[:, pl.ds(j * TN, TN)]), (cv, ci))
        cmin_s[:, pl.ds(j * TN, TN)] = mv
        cgid_s[:, pl.ds(j * TN, TN)] = mi

    @pl.when(i == NIB - 1)
    def _():
        vals = cmin_s[:, pl.ds(j * TN, TN)]
        frow = (cgid_s[:, pl.ds(j * TN, TN)] * 8
                + lax.broadcasted_iota(jnp.int32, (8, TN), 0))
        best = jnp.min(vals, axis=0, keepdims=True)
        cand = jnp.where(vals == best, frow, _IBIG)
        ynn_ref[...] = jnp.min(cand, axis=0).reshape(1, TN)


_tc_search = pl.pallas_call(
    _search_body,
    grid=(NIB, NJB),
    in_specs=[
        pl.BlockSpec((TM, 8), lambda i, j: (i, 0)),
        pl.BlockSpec((8, TN), lambda i, j: (0, j)),
        pl.BlockSpec((1, TN), lambda i, j: (0, j)),
        pl.BlockSpec((TM, 1), lambda i, j: (i, 0)),
    ],
    out_specs=(
        pl.BlockSpec((TM, 128), lambda i, j: (i, 0)),
        pl.BlockSpec((TM, 128), lambda i, j: (i, 0)),
        pl.BlockSpec((1, TN), lambda i, j: (0, j)),
    ),
    out_shape=(
        jax.ShapeDtypeStruct((N, 128), jnp.float32),
        jax.ShapeDtypeStruct((N, 128), jnp.int32),
        jax.ShapeDtypeStruct((1, M), jnp.int32),
    ),
    scratch_shapes=[
        pltpu.VMEM((8, M), jnp.float32),
        pltpu.VMEM((8, M), jnp.int32),
    ],
)


def _rowfin_body(rmin_ref, ridx_ref, xnn_ref):
    vals = rmin_ref[...]
    fidx = ridx_ref[...] * 128 + lax.broadcasted_iota(jnp.int32, (TM, 128), 1)
    best = jnp.min(vals, axis=1, keepdims=True)
    cand = jnp.where(vals == best, fidx, _IBIG)
    xnn_ref[...] = jnp.min(cand, axis=1).reshape(1, TM)


_tc_rowfin = pl.pallas_call(
    _rowfin_body,
    grid=(NIB,),
    in_specs=[
        pl.BlockSpec((TM, 128), lambda i: (i, 0)),
        pl.BlockSpec((TM, 128), lambda i: (i, 0)),
    ],
    out_specs=pl.BlockSpec((1, TM), lambda i: (0, i)),
    out_shape=jax.ShapeDtypeStruct((1, N), jnp.int32),
)


_SC_MESH = plsc.VectorSubcoreMesh(
    core_axis_name="c", subcore_axis_name="s",
    num_cores=NC, num_subcores=NS,
)


def _gather_body(xt_hbm, yt_hbm, xnn_hbm, ynn_hbm, rsum_hbm, csum_hbm,
                 xtv, ytv, ixv, iyv, rsv, csv):
    wid = lax.axis_index("s") * NC + lax.axis_index("c")
    base = wid * QPW
    pltpu.sync_copy(xt_hbm, xtv)
    pltpu.sync_copy(yt_hbm, ytv)
    pltpu.sync_copy(xnn_hbm.at[pl.ds(base, QPW)], ixv)
    pltpu.sync_copy(ynn_hbm.at[pl.ds(base, QPW)], iyv)

    def group(k, acc):
        accr, accc = acc
        o = k * L
        jn = ixv[pl.ds(o, L)]
        g0 = plsc.load_gather(ytv, [jn])
        g1 = plsc.load_gather(ytv, [jn + M])
        g2 = plsc.load_gather(ytv, [jn + 2 * M])
        x0 = xtv[pl.ds(base + o, L)]
        x1 = xtv[pl.ds(N + base + o, L)]
        x2 = xtv[pl.ds(2 * N + base + o, L)]
        d0 = x0 - g0
        d1 = x1 - g1
        d2 = x2 - g2
        accr = accr + (d0 * d0 + d1 * d1 + d2 * d2)
        im = iyv[pl.ds(o, L)]
        h0 = plsc.load_gather(xtv, [im])
        h1 = plsc.load_gather(xtv, [im + N])
        h2 = plsc.load_gather(xtv, [im + 2 * N])
        y0 = ytv[pl.ds(base + o, L)]
        y1 = ytv[pl.ds(M + base + o, L)]
        y2 = ytv[pl.ds(2 * M + base + o, L)]
        e0 = y0 - h0
        e1 = y1 - h1
        e2 = y2 - h2
        accc = accc + (e0 * e0 + e1 * e1 + e2 * e2)
        return accr, accc

    zero = jnp.zeros((L,), jnp.float32)
    accr, accc = lax.fori_loop(0, QPW // L, group, (zero, zero))
    rsv[...] = accr
    csv[...] = accc
    pltpu.sync_copy(rsv, rsum_hbm.at[wid])
    pltpu.sync_copy(csv, csum_hbm.at[wid])


_sc_gather = pl.kernel(
    _gather_body,
    out_type=(
        jax.ShapeDtypeStruct((W, L), jnp.float32),
        jax.ShapeDtypeStruct((W, L), jnp.float32),
    ),
    mesh=_SC_MESH,
    scratch_types=[
        pltpu.VMEM((3 * N,), jnp.float32),
        pltpu.VMEM((3 * M,), jnp.float32),
        pltpu.VMEM((QPW,), jnp.int32),
        pltpu.VMEM((QPW,), jnp.int32),
        pltpu.VMEM((L,), jnp.float32),
        pltpu.VMEM((L,), jnp.float32),
    ],
    compiler_params=pltpu.CompilerParams(needs_layout_passes=False),
)


def _finish_body(rs_ref, cs_ref, out_ref):
    out_ref[0, 0] = 0.5 * (jnp.sum(rs_ref[...]) / N + jnp.sum(cs_ref[...]) / M)


_tc_finish = pl.pallas_call(
    _finish_body,
    out_shape=jax.ShapeDtypeStruct((1, 1), jnp.float32),
    out_specs=pl.BlockSpec(memory_space=pltpu.MemorySpace.SMEM),
)


@jax.jit
def kernel(x, y):
    nx = jnp.sum(x * x, axis=1)
    ny = jnp.sum(y * y, axis=1)
    x8 = jnp.concatenate([x, jnp.zeros((N, 5), jnp.float32)], axis=1)
    y8t = jnp.concatenate([y, jnp.zeros((M, 5), jnp.float32)], axis=1).T
    rmin, ridx, ynn = _tc_search(x8, y8t, (0.5 * ny).reshape(1, M),
                                 (0.5 * nx).reshape(N, 1))
    xnn = _tc_rowfin(rmin, ridx)
    rsum, csum = _sc_gather(x.T.reshape(3 * N), y.T.reshape(3 * M),
                            xnn.reshape(N), ynn.reshape(M))
    out = _tc_finish(rsum, csum)
    return out[0, 0]


# speedup vs baseline: 3.9560x; 1.2493x over previous
import jax
import jax.numpy as jnp
from jax import lax
from jax.experimental import pallas as pl
from jax.experimental.pallas import tpu as pltpu
from jax.experimental.pallas import tpu_sc as plsc

N = 8192
M = 8192
L = 16
NC = 2
NS = 16
W = NC * NS
QPW = N // W

TM = 256
TN = 8192
NIB = N // TM
NJB = M // TN

_INF = float("inf")


NCH = TN // 128
NRS = TM // 8
_IBIG = 1 << 30


def _lexmin(a, b):
    av, ai = a
    bv, bi = b
    t = bv < av
    return jnp.where(t, bv, av), jnp.where(t, bi, ai)


def _tree_fold(items):
    while len(items) > 1:
        nxt = [_lexmin(items[k], items[k + 1])
               for k in range(0, len(items) - 1, 2)]
        if len(items) % 2:
            nxt.append(items[-1])
        items = nxt
    return items[0]


def _search_body(x8_ref, y8t_ref, nyh_ref, nxh_ref, rmin_o, ridx_o, ynn_ref,
                 cmin_s, cgid_s):
    i = pl.program_id(0)
    j = pl.program_id(1)

    xb = x8_ref[...].astype(jnp.bfloat16)
    yb = y8t_ref[...].astype(jnp.bfloat16)
    g = jnp.dot(xb, yb, preferred_element_type=jnp.float32)
    nyh = nyh_ref[...]
    nxh = nxh_ref[...]

    ritems = [(nyh[:, c * 128:(c + 1) * 128] - g[:, c * 128:(c + 1) * 128],
               jnp.full((TM, 128), j * NCH + c, jnp.int32))
              for c in range(NCH)]
    rv, ri = _tree_fold(ritems)

    @pl.when(j == 0)
    def _():
        rmin_o[...] = rv
        ridx_o[...] = ri

    @pl.when(j > 0)
    def _():
        mv, mi = _lexmin((rmin_o[...], ridx_o[...]), (rv, ri))
        rmin_o[...] = mv
        ridx_o[...] = mi

    citems = [(nxh[t * 8:(t + 1) * 8, :] - g[t * 8:(t + 1) * 8, :],
               jnp.full((8, TN), i * NRS + t, jnp.int32))
              for t in range(NRS)]
    cv, ci = _tree_fold(citems)

    @pl.when(i == 0)
    def _():
        cmin_s[:, pl.ds(j * TN, TN)] = cv
        cgid_s[:, pl.ds(j * TN, TN)] = ci

    @pl.when(i > 0)
    def _():
        mv, mi = _lexmin((cmin_s[:, pl.ds(j * TN, TN)],
                          cgid_s[:, pl.ds(j * TN, TN)]), (cv, ci))
        cmin_s[:, pl.ds(j * TN, TN)] = mv
        cgid_s[:, pl.ds(j * TN, TN)] = mi

    @pl.when(i == NIB - 1)
    def _():
        vals = cmin_s[:, pl.ds(j * TN, TN)]
        frow = (cgid_s[:, pl.ds(j * TN, TN)] * 8
                + lax.broadcasted_iota(jnp.int32, (8, TN), 0))
        best = jnp.min(vals, axis=0, keepdims=True)
        cand = jnp.where(vals == best, frow, _IBIG)
        ynn_ref[...] = jnp.min(cand, axis=0).reshape(1, TN)


_tc_search = pl.pallas_call(
    _search_body,
    grid=(NIB, NJB),
    in_specs=[
        pl.BlockSpec((TM, 8), lambda i, j: (i, 0)),
        pl.BlockSpec((8, TN), lambda i, j: (0, j)),
        pl.BlockSpec((1, TN), lambda i, j: (0, j)),
        pl.BlockSpec((TM, 1), lambda i, j: (i, 0)),
    ],
    out_specs=(
        pl.BlockSpec((TM, 128), lambda i, j: (i, 0)),
        pl.BlockSpec((TM, 128), lambda i, j: (i, 0)),
        pl.BlockSpec((1, TN), lambda i, j: (0, j)),
    ),
    out_shape=(
        jax.ShapeDtypeStruct((N, 128), jnp.float32),
        jax.ShapeDtypeStruct((N, 128), jnp.int32),
        jax.ShapeDtypeStruct((1, M), jnp.int32),
    ),
    scratch_shapes=[
        pltpu.VMEM((8, M), jnp.float32),
        pltpu.VMEM((8, M), jnp.int32),
    ],
)


def _rowfin_body(rmin_ref, ridx_ref, xnn_ref):
    vals = rmin_ref[...]
    fidx = ridx_ref[...] * 128 + lax.broadcasted_iota(jnp.int32, (TM, 128), 1)
    best = jnp.min(vals, axis=1, keepdims=True)
    cand = jnp.where(vals == best, fidx, _IBIG)
    xnn_ref[...] = jnp.min(cand, axis=1).reshape(1, TM)


_tc_rowfin = pl.pallas_call(
    _rowfin_body,
    grid=(NIB,),
    in_specs=[
        pl.BlockSpec((TM, 128), lambda i: (i, 0)),
        pl.BlockSpec((TM, 128), lambda i: (i, 0)),
    ],
    out_specs=pl.BlockSpec((1, TM), lambda i: (0, i)),
    out_shape=jax.ShapeDtypeStruct((1, N), jnp.int32),
)


_SC_MESH = plsc.VectorSubcoreMesh(
    core_axis_name="c", subcore_axis_name="s",
    num_cores=NC, num_subcores=NS,
)


def _gather_body(xt_hbm, yt_hbm, xnn_hbm, ynn_hbm, rsum_hbm, csum_hbm,
                 xtv, ytv, ixv, iyv, rsv, csv):
    wid = lax.axis_index("s") * NC + lax.axis_index("c")
    base = wid * QPW
    pltpu.sync_copy(xt_hbm, xtv)
    pltpu.sync_copy(yt_hbm, ytv)
    pltpu.sync_copy(xnn_hbm.at[pl.ds(base, QPW)], ixv)
    pltpu.sync_copy(ynn_hbm.at[pl.ds(base, QPW)], iyv)

    def group(k, acc):
        accr, accc = acc
        o = k * L
        jn = ixv[pl.ds(o, L)]
        g0 = plsc.load_gather(ytv, [jn])
        g1 = plsc.load_gather(ytv, [jn + M])
        g2 = plsc.load_gather(ytv, [jn + 2 * M])
        x0 = xtv[pl.ds(base + o, L)]
        x1 = xtv[pl.ds(N + base + o, L)]
        x2 = xtv[pl.ds(2 * N + base + o, L)]
        d0 = x0 - g0
        d1 = x1 - g1
        d2 = x2 - g2
        accr = accr + (d0 * d0 + d1 * d1 + d2 * d2)
        im = iyv[pl.ds(o, L)]
        h0 = plsc.load_gather(xtv, [im])
        h1 = plsc.load_gather(xtv, [im + N])
        h2 = plsc.load_gather(xtv, [im + 2 * N])
        y0 = ytv[pl.ds(base + o, L)]
        y1 = ytv[pl.ds(M + base + o, L)]
        y2 = ytv[pl.ds(2 * M + base + o, L)]
        e0 = y0 - h0
        e1 = y1 - h1
        e2 = y2 - h2
        accc = accc + (e0 * e0 + e1 * e1 + e2 * e2)
        return accr, accc

    zero = jnp.zeros((L,), jnp.float32)
    accr, accc = lax.fori_loop(0, QPW // L, group, (zero, zero))
    rsv[...] = accr
    csv[...] = accc
    pltpu.sync_copy(rsv, rsum_hbm.at[wid])
    pltpu.sync_copy(csv, csum_hbm.at[wid])


_sc_gather = pl.kernel(
    _gather_body,
    out_type=(
        jax.ShapeDtypeStruct((W, L), jnp.float32),
        jax.ShapeDtypeStruct((W, L), jnp.float32),
    ),
    mesh=_SC_MESH,
    scratch_types=[
        pltpu.VMEM((3 * N,), jnp.float32),
        pltpu.VMEM((3 * M,), jnp.float32),
        pltpu.VMEM((QPW,), jnp.int32),
        pltpu.VMEM((QPW,), jnp.int32),
        pltpu.VMEM((L,), jnp.float32),
        pltpu.VMEM((L,), jnp.float32),
    ],
    compiler_params=pltpu.CompilerParams(needs_layout_passes=False),
)


def _finish_body(rs_ref, cs_ref, out_ref):
    out_ref[0, 0] = 0.5 * (jnp.sum(rs_ref[...]) / N + jnp.sum(cs_ref[...]) / M)


_tc_finish = pl.pallas_call(
    _finish_body,
    out_shape=jax.ShapeDtypeStruct((1, 1), jnp.float32),
    out_specs=pl.BlockSpec(memory_space=pltpu.MemorySpace.SMEM),
)


@jax.jit
def kernel(x, y):
    nx = jnp.sum(x * x, axis=1)
    ny = jnp.sum(y * y, axis=1)
    x8 = jnp.concatenate([x, jnp.zeros((N, 5), jnp.float32)], axis=1)
    y8t = jnp.concatenate([y, jnp.zeros((M, 5), jnp.float32)], axis=1).T
    rmin, ridx, ynn = _tc_search(x8, y8t, (0.5 * ny).reshape(1, M),
                                 (0.5 * nx).reshape(N, 1))
    xnn = _tc_rowfin(rmin, ridx)
    rsum, csum = _sc_gather(x.T.reshape(3 * N), y.T.reshape(3 * M),
                            xnn.reshape(N), ynn.reshape(M))
    out = _tc_finish(rsum, csum)
    return out[0, 0]
